# baseline (speedup 1.0000x reference)
import functools

import jax
import jax.numpy as jnp
from jax import lax
from jax.experimental import pallas as pl
from jax.experimental.pallas import tpu as pltpu
from jax.experimental.pallas import tpu_sc as plsc

VOCAB = 1000000
DIM = 64
BATCH = 16384
HIST = 50

BLK = 128
NBLK = BATCH * HIST // BLK
NW = 32
BLK_PW = NBLK // NW
NBUF = 4
T = BLK_PW // NBUF
DT = DIM // 8
BT = BATCH // 128


def _embed_body(idx_hbm, table_hbm, out_hbm, idx_v, rows_v, tp_v,
                gs0, gs1, gs2, gs3, os0, os1, os2, os3):
    gsems = (gs0, gs1, gs2, gs3)
    osems = (os0, os1, os2, os3)
    cid = lax.axis_index("c")
    sid = lax.axis_index("s")
    wid = sid * 2 + cid
    pltpu.sync_copy(idx_hbm.at[pl.ds(wid * BLK_PW, BLK_PW)], idx_v)
    g0 = wid * BLK_PW

    def gfire(l, j):
        pltpu.async_copy(table_hbm.at[idx_v.at[l]], rows_v.at[j], gsems[j])

    def gwait(l, j):
        pltpu.make_async_copy(
            table_hbm.at[idx_v.at[l]], rows_v.at[j], gsems[j]).wait()

    def ocopy(l, j):
        g = g0 + l
        h = g // 128
        bt = g % 128
        return pltpu.make_async_copy(
            tp_v.at[j, :, :, :, pl.ds(0, 128)],
            out_hbm.at[pl.ds(h * DT, DT), pl.ds(bt, 1)],
            osems[j],
        )

    lanes = jnp.arange(16, dtype=jnp.int32)
    dt_base = lanes // 8
    r_base = lanes % 8
    zv = jnp.zeros((16,), dtype=jnp.int32)
    d0s = tuple(range(0, DIM, 16))

    def transpose(j):
        @plsc.parallel_loop(0, BLK, step=1, unroll=16)
        def _(c):
            cv = zv + c
            vs = [rows_v[j, c, pl.ds(d0, 16)] for d0 in d0s]
            for k, d0 in enumerate(d0s):
                plsc.store_scatter(
                    tp_v.at[j],
                    [dt_base + (d0 // 8), zv, r_base, cv], vs[k])

    for j in range(NBUF):
        gfire(j, j)

    def ring(t, carry):
        for j in range(NBUF):
            l = t * NBUF + j
            gwait(l, j)

            @pl.when(t > 0)
            def _():
                ocopy(l - NBUF, j).wait()

            transpose(j)
            ocopy(l, j).start()

            @pl.when(t < T - 1)
            def _():
                gfire(l + NBUF, j)

        return carry

    lax.fori_loop(0, T, ring, 0)

    for j in range(NBUF):
        ocopy(BLK_PW - NBUF + j, j).wait()


@functools.partial(jax.jit, static_argnames=())
def kernel(input_, table):
    idx = input_.T.reshape(NBLK, BLK).astype(jnp.int32)
    mesh = plsc.VectorSubcoreMesh(core_axis_name="c", subcore_axis_name="s")
    out = pl.kernel(
        _embed_body,
        out_type=jax.ShapeDtypeStruct((HIST * DT, BT, 8, 128), jnp.float32),
        mesh=mesh,
        scratch_types=[
            pltpu.VMEM((BLK_PW, BLK), jnp.int32),
            pltpu.VMEM((NBUF, BLK, DIM), jnp.float32),
            pltpu.VMEM((NBUF, DT, 1, 8, 129), jnp.float32),
        ] + [pltpu.SemaphoreType.DMA] * (2 * NBUF),
        compiler_params=pltpu.CompilerParams(
            use_tc_tiling_on_sc=False, needs_layout_passes=False),
    )(idx, table)
    out = out.reshape(HIST, DT, BT, 8, 128)
    return out.transpose(2, 4, 0, 1, 3).reshape(BATCH, HIST, DIM)


# speedup vs baseline: 1.0047x; 1.0047x over previous
import functools

import jax
import jax.numpy as jnp
from jax import lax
from jax.experimental import pallas as pl
from jax.experimental.pallas import tpu as pltpu
from jax.experimental.pallas import tpu_sc as plsc

VOCAB = 1000000
DIM = 64
BATCH = 16384
HIST = 50

BLK = 128
NBLK = BATCH * HIST // BLK
NW = 32
BLK_PW = NBLK // NW
NBUF = 5
T = BLK_PW // NBUF
DT = DIM // 8
BT = BATCH // 128


def _embed_body(idx_hbm, table_hbm, out_hbm, idx_v, rows_v, tp_v,
                gs0, gs1, gs2, gs3, gs4, os0, os1, os2, os3, os4):
    gsems = (gs0, gs1, gs2, gs3, gs4)
    osems = (os0, os1, os2, os3, os4)
    cid = lax.axis_index("c")
    sid = lax.axis_index("s")
    wid = sid * 2 + cid
    pltpu.sync_copy(idx_hbm.at[pl.ds(wid * BLK_PW, BLK_PW)], idx_v)
    g0 = wid * BLK_PW

    def gfire(l, j):
        pltpu.async_copy(table_hbm.at[idx_v.at[l]], rows_v.at[j], gsems[j])

    def gwait(l, j):
        pltpu.make_async_copy(
            table_hbm.at[idx_v.at[l]], rows_v.at[j], gsems[j]).wait()

    def ocopy(l, j):
        g = g0 + l
        h = g // 128
        bt = g % 128
        return pltpu.make_async_copy(
            tp_v.at[j, :, :, :, pl.ds(0, 128)],
            out_hbm.at[pl.ds(h * DT, DT), pl.ds(bt, 1)],
            osems[j],
        )

    lanes = jnp.arange(16, dtype=jnp.int32)
    dt_base = lanes // 8
    r_base = lanes % 8
    zv = jnp.zeros((16,), dtype=jnp.int32)
    d0s = tuple(range(0, DIM, 16))

    def transpose(j):
        @plsc.parallel_loop(0, BLK, step=1, unroll=8)
        def _(c):
            cv = zv + c
            vs = [rows_v[j, c, pl.ds(d0, 16)] for d0 in d0s]
            for k, d0 in enumerate(d0s):
                plsc.store_scatter(
                    tp_v.at[j],
                    [dt_base + (d0 // 8), zv, r_base, cv], vs[k])

    for j in range(NBUF):
        gfire(j, j)

    def ring(t, carry):
        for j in range(NBUF):
            l = t * NBUF + j
            gwait(l, j)

            @pl.when(t > 0)
            def _():
                ocopy(l - NBUF, j).wait()

            transpose(j)
            ocopy(l, j).start()

            @pl.when(t < T - 1)
            def _():
                gfire(l + NBUF, j)

        return carry

    lax.fori_loop(0, T, ring, 0)

    for j in range(NBUF):
        ocopy(BLK_PW - NBUF + j, j).wait()


@functools.partial(jax.jit, static_argnames=())
def kernel(input_, table):
    idx = input_.T.reshape(NBLK, BLK).astype(jnp.int32)
    mesh = plsc.VectorSubcoreMesh(core_axis_name="c", subcore_axis_name="s")
    out = pl.kernel(
        _embed_body,
        out_type=jax.ShapeDtypeStruct((HIST * DT, BT, 8, 128), jnp.float32),
        mesh=mesh,
        scratch_types=[
            pltpu.VMEM((BLK_PW, BLK), jnp.int32),
            pltpu.VMEM((NBUF, BLK, DIM), jnp.float32),
            pltpu.VMEM((NBUF, DT, 1, 8, 129), jnp.float32),
        ] + [pltpu.SemaphoreType.DMA] * (2 * NBUF),
        compiler_params=pltpu.CompilerParams(
            use_tc_tiling_on_sc=False, needs_layout_passes=False),
    )(idx, table)
    out = out.reshape(HIST, DT, BT, 8, 128)
    return out.transpose(2, 4, 0, 1, 3).reshape(BATCH, HIST, DIM)
